# baseline (speedup 1.0000x reference)
import functools

import jax
import jax.numpy as jnp
from jax import lax
from jax.experimental import pallas as pl
from jax.experimental.pallas import tpu as pltpu
from jax.experimental.pallas import tpu_sc as plsc

_NC = 2
_NS = 16
_NW = _NC * _NS
_LANES = 16


@functools.lru_cache(maxsize=None)
def _make_sc_add(batch: int, flat: int):
    assert flat % (_NW * _LANES) == 0
    chunk = flat // _NW
    nvec = chunk // _LANES
    mesh = plsc.VectorSubcoreMesh(core_axis_name="c", subcore_axis_name="s")

    def body(x_hbm, pos_hbm, out_hbm, pos_v, buf0, buf1,
             in_sem0, in_sem1, out_sem0, out_sem1):
        wid = lax.axis_index("s") * _NC + lax.axis_index("c")
        base = wid * chunk
        bufs = (buf0, buf1)
        in_sems = (in_sem0, in_sem1)
        out_sems = (out_sem0, out_sem1)
        in_cp = [None, None]
        out_cp = [None, None]

        def start_in(b, k):
            in_cp[k] = pltpu.async_copy(
                x_hbm.at[b, pl.ds(base, chunk)], bufs[k], in_sems[k])

        start_in(0, 0)
        pltpu.sync_copy(pos_hbm.at[pl.ds(base, chunk)], pos_v)

        for b in range(batch):
            k = b & 1
            if b + 1 < batch:
                if b >= 1:
                    out_cp[1 - k].wait()
                start_in(b + 1, 1 - k)
            in_cp[k].wait()
            buf = bufs[k]

            @plsc.parallel_loop(0, nvec, 1, unroll=8)
            def _(i):
                sl = pl.ds(i * _LANES, _LANES)
                plsc.addupdate(buf.at[sl], pos_v[sl])

            out_cp[k] = pltpu.async_copy(
                buf, out_hbm.at[b, pl.ds(base, chunk)], out_sems[k])

        out_cp[0].wait()
        out_cp[1].wait()

    return pl.kernel(
        body,
        out_type=jax.ShapeDtypeStruct((batch, flat), jnp.float32),
        mesh=mesh,
        scratch_types=[
            pltpu.VMEM((chunk,), jnp.float32),
            pltpu.VMEM((chunk,), jnp.float32),
            pltpu.VMEM((chunk,), jnp.float32),
            pltpu.SemaphoreType.DMA,
            pltpu.SemaphoreType.DMA,
            pltpu.SemaphoreType.DMA,
            pltpu.SemaphoreType.DMA,
        ],
    )


@jax.jit
def kernel(x, pos_embedding):
    batch, g, h, w, d = x.shape
    flat = g * h * w * d
    x_flat = x.reshape(batch, flat)
    pos_flat = pos_embedding.reshape(-1)
    out = _make_sc_add(batch, flat)(x_flat, pos_flat)
    return out.reshape(x.shape)


# speedup vs baseline: 1.9760x; 1.9760x over previous
import functools

import jax
import jax.numpy as jnp
from jax import lax
from jax.experimental import pallas as pl
from jax.experimental.pallas import tpu as pltpu
from jax.experimental.pallas import tpu_sc as plsc

_NC = 2
_NS = 16
_NW = _NC * _NS
_LANES = 16


@functools.lru_cache(maxsize=None)
def _make_sc_add(batch: int, seq: int, d: int, pos_rows: int):
    assert seq % _NW == 0
    rows = seq // _NW
    nvec = rows * d // _LANES
    npl = d // _LANES
    mesh = plsc.VectorSubcoreMesh(core_axis_name="c", subcore_axis_name="s")

    def body(x_hbm, pos_hbm, out_hbm, pos_v, buf0, buf1,
             in_sem0, in_sem1, out_sem0, out_sem1):
        wid = lax.axis_index("s") * _NC + lax.axis_index("c")
        base = wid * rows
        bufs = (buf0, buf1)
        in_sems = (in_sem0, in_sem1)
        out_sems = (out_sem0, out_sem1)
        in_cp = [None, None]
        out_cp = [None, None]

        def start_in(b, k):
            in_cp[k] = pltpu.async_copy(
                x_hbm.at[b, pl.ds(base, rows)], bufs[k], in_sems[k])

        start_in(0, 0)
        pltpu.sync_copy(pos_hbm.at[pl.ds(base, rows)], pos_v)

        for b in range(batch):
            k = b & 1
            if b + 1 < batch:
                if b >= 1:
                    out_cp[1 - k].wait()
                start_in(b + 1, 1 - k)
            in_cp[k].wait()
            buf = bufs[k]

            @plsc.parallel_loop(0, nvec, 1, unroll=8)
            def _(i):
                r = i // npl
                sl = pl.ds((i % npl) * _LANES, _LANES)
                plsc.addupdate(buf.at[r, sl], pos_v[r, sl])

            out_cp[k] = pltpu.async_copy(
                buf, out_hbm.at[b, pl.ds(base, rows)], out_sems[k])

        out_cp[0].wait()
        out_cp[1].wait()

    return pl.kernel(
        body,
        out_type=jax.ShapeDtypeStruct((batch, seq, d), jnp.float32),
        mesh=mesh,
        scratch_types=[
            pltpu.VMEM((rows, d), jnp.float32),
            pltpu.VMEM((rows, d), jnp.float32),
            pltpu.VMEM((rows, d), jnp.float32),
            pltpu.SemaphoreType.DMA,
            pltpu.SemaphoreType.DMA,
            pltpu.SemaphoreType.DMA,
            pltpu.SemaphoreType.DMA,
        ],
    )


@jax.jit
def kernel(x, pos_embedding):
    batch, g, h, w, d = x.shape
    seq = g * h * w
    x3 = x.reshape(batch, seq, d)
    out = _make_sc_add(batch, seq, d, pos_embedding.shape[0])(x3, pos_embedding)
    return out.reshape(x.shape)
